# baseline (speedup 1.0000x reference)
import functools

import jax
import jax.numpy as jnp
from jax import lax
from jax.experimental import pallas as pl
from jax.experimental.pallas import tpu as pltpu
from jax.experimental.pallas import tpu_sc as plsc

N = 10000
E = 320000
D = 128
DE = 16
NL = 3
DELTA = 2.5

NB = 64
NPB = 160
NPAD = 10304
REG = 10560
NT = 32
EPT = E // NT
EP = NT * REG
CH = 96
CHL = 96
BINMUL = 52429

MESH = plsc.VectorSubcoreMesh(core_axis_name="c", subcore_axis_name="s")
SC_PARAMS = pltpu.CompilerParams(needs_layout_passes=False)


def _wid():
    return lax.axis_index("s") * 2 + lax.axis_index("c")


def _iota():
    return lax.iota(jnp.int32, 16)


def _extract(vec16):
    return lax.reduce_max(vec16, (0,))


def _bin_of(d):
    return lax.shift_right_logical(d * BINMUL, 23)


@functools.partial(
    pl.kernel, mesh=MESH, compiler_params=SC_PARAMS,
    out_type=(jax.ShapeDtypeStruct((EP,), jnp.int32),
              jax.ShapeDtypeStruct((EP,), jnp.int32),
              jax.ShapeDtypeStruct((EP,), jnp.int32),
              jax.ShapeDtypeStruct((NT * NB,), jnp.int32),
              jax.ShapeDtypeStruct((NT * NB,), jnp.int32),
              jax.ShapeDtypeStruct((2, NPAD), jnp.float32)),
    scratch_types=[
        pltpu.VMEM((400,), jnp.int32),
        pltpu.VMEM((128,), jnp.int32),
        pltpu.VMEM((128,), jnp.int32),
        pltpu.VMEM((128,), jnp.int32),
        pltpu.VMEM((REG,), jnp.int32),
        pltpu.VMEM((CH,), jnp.int32),
        pltpu.VMEM((CH,), jnp.int32),
        pltpu.VMEM((CH,), jnp.int32),
        pltpu.VMEM((CH,), jnp.int32),
        pltpu.VMEM((CH,), jnp.int32),
        pltpu.VMEM((CH,), jnp.float32),
        pltpu.VMEM_SHARED((NPAD,), jnp.float32),
        pltpu.SemaphoreType.DMA,
        pltpu.SemaphoreType.DMA,
        pltpu.SemaphoreType.DMA,
    ],
)
def _bin_kernel(srcP, dstP, zerosN,
                srcb, dstlocb, eidcb, cnts, loffs, degp,
                dstv, hist, curs, tagv, eidb, idx64, st_src, st_dst, st_dl,
                st_di, ones, deg_sh, sem0, sem1, sem2):
    w = _wid()
    cid = lax.axis_index("c")
    it = _iota()
    ebase = w * EPT

    @pl.when(w < 2)
    def _zero_deg():
        pltpu.sync_copy(zerosN, deg_sh)
    plsc.subcore_barrier()

    for g in range(8):
        hist[pl.ds(g * 16, 16)] = jnp.zeros((16,), jnp.int32)
    one16 = jnp.full((16,), 1, jnp.int32)

    def hist_body(i, _):
        start = pl.multiple_of(ebase + i * 400, 8)
        pltpu.sync_copy(dstP.at[pl.ds(start, 400)], dstv)
        for g in range(25):
            d16 = dstv[pl.ds(g * 16, 16)]
            plsc.addupdate_scatter(hist, [_bin_of(d16)], one16)
        return 0
    lax.fori_loop(0, 25, hist_body, 0)

    carry = jnp.int32(0)
    for g in range(4):
        hv = hist[pl.ds(g * 16, 16)]
        c8 = jnp.bitwise_and(hv + 7, -8)
        incl = plsc.cumsum(c8)
        curs[pl.ds(g * 16, 16)] = incl - c8 + carry
        carry = carry + jnp.sum(c8)
    for g in range(4, 8):
        curs[pl.ds(g * 16, 16)] = jnp.zeros((16,), jnp.int32)
    pltpu.sync_copy(hist.at[pl.ds(0, NB)], cnts.at[pl.ds(pl.multiple_of(w * NB, 8), NB)])
    pltpu.sync_copy(curs.at[pl.ds(0, NB)], loffs.at[pl.ds(pl.multiple_of(w * NB, 8), NB)])

    sent16 = jnp.full((16,), E, jnp.int32)

    def fill_body(i, _):
        eidb[pl.ds(pl.multiple_of(i * 16, 16), 16)] = sent16
        return 0
    lax.fori_loop(0, REG // 16, fill_body, 0)

    def perm_body(i, _):
        start = pl.multiple_of(ebase + i * 400, 8)
        pltpu.sync_copy(dstP.at[pl.ds(start, 400)], dstv)
        for g in range(25):
            d16 = dstv[pl.ds(g * 16, 16)]
            b16 = _bin_of(d16)
            eid16 = jnp.full((16,), ebase + i * 400 + g * 16, jnp.int32) + it

            def cond(c):
                return _extract(plsc.all_reduce_population_count(c[0])) > 0

            def body(c):
                pending = c[0]
                plsc.store_scatter(tagv, [b16], it, mask=pending)
                seen = plsc.load_gather(tagv, [b16])
                win = jnp.logical_and(pending, seen == it)
                cur = plsc.load_gather(curs, [b16])
                plsc.store_scatter(eidb, [cur], eid16, mask=win)
                plsc.addupdate_scatter(curs, [b16], one16, mask=win)
                return (jnp.logical_and(pending, jnp.logical_not(win)), 0)
            lax.while_loop(cond, body, (jnp.full((16,), True), 0))
        return 0
    lax.fori_loop(0, 25, perm_body, 0)

    for g in range(CH // 16):
        ones[pl.ds(g * 16, 16)] = jnp.full((16,), 1.0, jnp.float32)

    def out_body(j, _):
        jo = pl.multiple_of(j * CH, 8)
        for g in range(CH // 16):
            raw = eidb[pl.ds(jo + g * 16, 16)]
            idx64[pl.ds(g * 16, 16)] = jnp.minimum(raw, E - 1)
        cp0 = pltpu.async_copy(srcP.at[idx64], st_src, sem0)
        cp1 = pltpu.async_copy(dstP.at[idx64], st_dst, sem1)
        cp0.wait()
        cp1.wait()
        for g in range(CH // 16):
            sl = pl.ds(g * 16, 16)
            d16 = st_dst[sl]
            pad = eidb[pl.ds(jo + g * 16, 16)] == E
            dl = d16 - _bin_of(d16) * NPB
            st_dl[sl] = jnp.where(pad, NPB, dl)
            st_di[sl] = jnp.where(pad, NB * NPB + w, d16)
        pltpu.sync_copy(ones, deg_sh.at[st_di], add=True)
        obase = pl.multiple_of(w * REG + j * CH, 8)
        pltpu.sync_copy(st_src, srcb.at[pl.ds(obase, CH)])
        pltpu.sync_copy(st_dl, dstlocb.at[pl.ds(obase, CH)])
        pltpu.sync_copy(idx64, eidcb.at[pl.ds(obase, CH)])
        return 0
    lax.fori_loop(0, REG // CH, out_body, 0)

    plsc.subcore_barrier()

    @pl.when(w < 2)
    def _deg_out():
        pltpu.sync_copy(deg_sh, degp.at[cid])


ACC = (NPB + 1) * D


def _make_layer_kernel(last):
    nout = 2 if last else 4

    @functools.partial(
        pl.kernel, mesh=MESH, compiler_params=SC_PARAMS,
        out_type=tuple(jax.ShapeDtypeStruct((NB * NPB * D,), jnp.float32)
                       for _ in range(nout)),
        scratch_types=[pltpu.VMEM((ACC,), jnp.float32)] * nout + [
            pltpu.VMEM((CHL, D), jnp.float32),
            pltpu.VMEM((CHL, D), jnp.float32),
            pltpu.VMEM((CHL, D), jnp.float32),
            pltpu.VMEM((CHL,), jnp.int32),
            pltpu.VMEM((CHL,), jnp.int32),
            pltpu.VMEM((CHL,), jnp.int32),
            pltpu.VMEM((CHL,), jnp.int32),
            pltpu.VMEM((NT * NB,), jnp.int32),
            pltpu.VMEM((NT * NB,), jnp.int32),
            pltpu.SemaphoreType.DMA,
            pltpu.SemaphoreType.DMA,
            pltpu.SemaphoreType.DMA,
        ],
    )
    def _lk(srcb, dstlocb, eidcb, etb, psrc, pdstP, cnts, loffs, *rest):
        outs = rest[:nout]
        accs = rest[nout:2 * nout]
        (st_ps, st_pd, st_et, srcv, dstlv, gdstv, eidv, tcnt, toff,
         sem0, sem1, sem2) = rest[2 * nout:]
        if last:
            (S, MX) = outs
            (accS, accMX) = accs
        else:
            (S, Q, MX, MN) = outs
            (accS, accQ, accMX, accMN) = accs
        w = _wid()
        it = _iota()
        zero16 = jnp.zeros((16,), jnp.float32)
        neg16 = jnp.full((16,), -3e38, jnp.float32)
        pos16 = jnp.full((16,), 3e38, jnp.float32)

        pltpu.sync_copy(cnts, tcnt)
        pltpu.sync_copy(loffs, toff)

        for b_i in range(2):
            b = w * 2 + b_i

            def init_body(i, _):
                sl = pl.ds(pl.multiple_of(i * 16, 16), 16)
                accS[sl] = zero16
                accMX[sl] = neg16
                if not last:
                    accQ[sl] = zero16
                    accMN[sl] = pos16
                return 0
            lax.fori_loop(0, ACC // 16, init_body, 0)

            def region_body(r, _):
                cnt_rb = _extract(plsc.load_gather(tcnt, [jnp.full((16,), r * NB, jnp.int32) + b]))
                off_rb = _extract(plsc.load_gather(toff, [jnp.full((16,), r * NB, jnp.int32) + b]))
                base = r * REG + off_rb

                def chunk_body(ch, _):
                    start = pl.multiple_of(base + ch * CHL, 8)
                    a0 = pltpu.async_copy(srcb.at[pl.ds(start, CHL)], srcv, sem0)
                    a1 = pltpu.async_copy(dstlocb.at[pl.ds(start, CHL)], dstlv, sem1)
                    a2 = pltpu.async_copy(eidcb.at[pl.ds(start, CHL)], eidv, sem2)
                    a0.wait()
                    a2.wait()
                    b0 = pltpu.async_copy(etb.at[eidv], st_et, sem0)
                    b1 = pltpu.async_copy(psrc.at[srcv], st_ps, sem2)
                    a1.wait()
                    for g in range(CHL // 16):
                        sl = pl.ds(g * 16, 16)
                        valid = (jnp.full((16,), ch * CHL + g * 16, jnp.int32) + it) < cnt_rb
                        dl = jnp.where(valid, dstlv[sl], NPB)
                        dstlv[sl] = dl
                        gdstv[sl] = dl + b * NPB
                    b2 = pltpu.async_copy(pdstP.at[gdstv], st_pd, sem1)
                    b0.wait()
                    b1.wait()
                    b2.wait()

                    def do_edge(e):
                        ev = jnp.full((16,), e, jnp.int32)
                        rowbase = plsc.load_gather(dstlv, [ev]) * D

                        @plsc.parallel_loop(0, 8, unroll=8)
                        def gbody(g):
                            colv = jnp.full((16,), g * 16, jnp.int32) + it
                            ps = plsc.load_gather(st_ps, [ev, colv])
                            pd = plsc.load_gather(st_pd, [ev, colv])
                            et = plsc.load_gather(st_et, [ev, colv])
                            m = jnp.maximum(ps + pd + et, 0.0)
                            aidx = rowbase + colv
                            plsc.store_scatter(accS, [aidx], plsc.load_gather(accS, [aidx]) + m)
                            plsc.store_scatter(accMX, [aidx], jnp.maximum(plsc.load_gather(accMX, [aidx]), m))
                            if not last:
                                plsc.store_scatter(accQ, [aidx], plsc.load_gather(accQ, [aidx]) + m * m)
                                plsc.store_scatter(accMN, [aidx], jnp.minimum(plsc.load_gather(accMN, [aidx]), m))

                    def edge_body(e2, _):
                        do_edge(e2 * 2)
                        do_edge(e2 * 2 + 1)
                        return 0
                    lax.fori_loop(0, CHL // 2, edge_body, 0)
                    return 0
                nch = lax.div(cnt_rb + (CHL - 1), CHL)
                lax.fori_loop(0, nch, chunk_body, 0)
                return 0
            lax.fori_loop(0, NT, region_body, 0)

            obase = pl.multiple_of(b * NPB * D, 8)
            for acc, out in zip(accs, outs):
                pltpu.sync_copy(acc.at[pl.ds(0, NPB * D)], out.at[pl.ds(obase, NPB * D)])
    return _lk


_layer_full = _make_layer_kernel(False)
_layer_last = _make_layer_kernel(True)


def _mm_k(a_ref, b_ref, o_ref):
    o_ref[...] = jnp.dot(a_ref[...], b_ref[...], preferred_element_type=jnp.float32)


def _mm(a, b, block_rows):
    m, k = a.shape
    _, n = b.shape
    return pl.pallas_call(
        _mm_k,
        grid=(m // block_rows,),
        in_specs=[pl.BlockSpec((block_rows, k), lambda i: (i, 0)),
                  pl.BlockSpec((k, n), lambda i: (0, 0))],
        out_specs=pl.BlockSpec((block_rows, n), lambda i: (i, 0)),
        out_shape=jax.ShapeDtypeStruct((m, n), jnp.float32),
    )(a, b)


def _post_k(last, x_ref, s_ref, q_ref, mx_ref, mn_ref, deg_ref, w_ref, bias_ref, o_ref):
    x = x_ref[...]
    s = s_ref[...]
    deg = deg_ref[...]
    degc = jnp.maximum(deg, 1.0)
    has = (deg > 0).astype(jnp.float32)
    logd = jnp.log(deg + 1.0)
    amp = logd / DELTA
    att = DELTA / jnp.maximum(logd, 1e-6)
    mean = s / degc
    mx = mx_ref[...] * has
    if last:
        agg = jnp.concatenate([mean, mx, s], axis=1)
        feats = jnp.concatenate([x, agg, agg * amp], axis=1)
    else:
        mn = mn_ref[...] * has
        q = q_ref[...] / degc
        std = jnp.sqrt(jax.nn.relu(q - mean * mean) + 1e-5)
        agg = jnp.concatenate([mean, mx, mn, std], axis=1)
        feats = jnp.concatenate([x, agg, agg * amp, agg * att], axis=1)
    o_ref[...] = x + jnp.dot(feats, w_ref[...], preferred_element_type=jnp.float32) + bias_ref[...]


def _post(x, s, q, mx, mn, deg, wpost, bpost, last):
    br = 1000
    kdim = wpost.shape[0]
    return pl.pallas_call(
        functools.partial(_post_k, last),
        grid=(N // br,),
        in_specs=[pl.BlockSpec((br, D), lambda i: (i, 0)),
                  pl.BlockSpec((br, D), lambda i: (i, 0)),
                  pl.BlockSpec((br, D), lambda i: (i, 0)),
                  pl.BlockSpec((br, D), lambda i: (i, 0)),
                  pl.BlockSpec((br, D), lambda i: (i, 0)),
                  pl.BlockSpec((br, 1), lambda i: (i, 0)),
                  pl.BlockSpec((kdim, D), lambda i: (0, 0)),
                  pl.BlockSpec((1, D), lambda i: (0, 0))],
        out_specs=pl.BlockSpec((br, D), lambda i: (i, 0)),
        out_shape=jax.ShapeDtypeStruct((N, D), jnp.float32),
    )(x, s, q, mx, mn, deg, wpost, bpost)


def _readout_k(x_ref, w0_ref, b0_ref, w1_ref, b1_ref, w2_ref, b2_ref, o_ref):
    hg = jnp.mean(x_ref[...], axis=0, keepdims=True)
    r = jax.nn.relu(jnp.dot(hg, w0_ref[...], preferred_element_type=jnp.float32) + b0_ref[...])
    r = jax.nn.relu(jnp.dot(r, w1_ref[...], preferred_element_type=jnp.float32) + b1_ref[...])
    o_ref[...] = jnp.dot(r, w2_ref[...], preferred_element_type=jnp.float32) + b2_ref[...]


def _readout(x, w0, b0, w1, b1, w2, b2):
    return pl.pallas_call(
        _readout_k,
        out_shape=jax.ShapeDtypeStruct((1, 2), jnp.float32),
    )(x, w0, b0.reshape(1, -1), w1, b1.reshape(1, -1), w2, b2.reshape(1, -1))


def _mmb_k(a_ref, b_ref, bias_ref, o_ref):
    o_ref[...] = jnp.dot(a_ref[...], b_ref[...], preferred_element_type=jnp.float32) + bias_ref[...]


def _mm_bias(a, b, bias, block_rows):
    m, k = a.shape
    _, n = b.shape
    return pl.pallas_call(
        _mmb_k,
        grid=(m // block_rows,),
        in_specs=[pl.BlockSpec((block_rows, k), lambda i: (i, 0)),
                  pl.BlockSpec((k, n), lambda i: (0, 0)),
                  pl.BlockSpec((1, n), lambda i: (0, 0))],
        out_specs=pl.BlockSpec((block_rows, n), lambda i: (i, 0)),
        out_shape=jax.ShapeDtypeStruct((m, n), jnp.float32),
    )(a, b, bias.reshape(1, -1))


def kernel(h, e, edge_index, W_h, b_h, W_e, b_e, pre_Ws, pre_bs, post_Ws,
           post_bs, Wr0, br0, Wr1, br1, Wr2, br2):
    src = edge_index[0].astype(jnp.int32)
    dst = edge_index[1].astype(jnp.int32)
    zerosN = jnp.zeros((NPAD,), jnp.float32)

    srcb, dstlocb, eidcb, cnts, loffs, degp = _bin_kernel(src, dst, zerosN)
    deg = (degp[0] + degp[1])[:N].reshape(N, 1)

    x = _mm(h, W_h, 2000) + b_h

    for l in range(NL):
        Wpre = pre_Ws[l]
        Ws, We2, Wd = Wpre[:D], Wpre[D:2 * D], Wpre[2 * D:]
        Me = W_e @ We2
        be = b_e @ We2 + pre_bs[l]
        Psrc = _mm(x, Ws, 2000)
        Pdst = _mm(x, Wd, 2000)
        PdstP = jnp.concatenate([Pdst, jnp.zeros((NPAD - N, D), jnp.float32)], axis=0)
        Et = _mm_bias(e, Me, be, 8000)
        if l < NL - 1:
            Sf, Qf, MXf, MNf = _layer_full(srcb, dstlocb, eidcb, Et, Psrc,
                                           PdstP, cnts, loffs)
        else:
            Sf, MXf = _layer_last(srcb, dstlocb, eidcb, Et, Psrc, PdstP,
                                  cnts, loffs)
            Qf, MNf = Sf, Sf
        S = Sf.reshape(NB * NPB, D)[:N]
        Q = Qf.reshape(NB * NPB, D)[:N]
        MX = MXf.reshape(NB * NPB, D)[:N]
        MN = MNf.reshape(NB * NPB, D)[:N]
        x = _post(x, S, Q, MX, MN, deg, post_Ws[l], post_bs[l].reshape(1, D),
                  last=(l == NL - 1))

    return _readout(x, Wr0, br0, Wr1, br1, Wr2, br2)


# speedup vs baseline: 1.0713x; 1.0713x over previous
import functools

import jax
import jax.numpy as jnp
from jax import lax
from jax.experimental import pallas as pl
from jax.experimental.pallas import tpu as pltpu
from jax.experimental.pallas import tpu_sc as plsc

N = 10000
E = 320000
D = 128
DE = 16
NL = 3
DELTA = 2.5

NB = 64
NPB = 160
NPAD = 10304
REG = 10560
NT = 32
EPT = E // NT
EP = NT * REG
CH = 96
CHL = 96
BINMUL = 52429

MESH = plsc.VectorSubcoreMesh(core_axis_name="c", subcore_axis_name="s")
SC_PARAMS = pltpu.CompilerParams(needs_layout_passes=False)


def _wid():
    return lax.axis_index("s") * 2 + lax.axis_index("c")


def _iota():
    return lax.iota(jnp.int32, 16)


def _extract(vec16):
    return lax.reduce_max(vec16, (0,))


def _bin_of(d):
    return lax.shift_right_logical(d * BINMUL, 23)


@functools.partial(
    pl.kernel, mesh=MESH, compiler_params=SC_PARAMS,
    out_type=(jax.ShapeDtypeStruct((EP,), jnp.int32),
              jax.ShapeDtypeStruct((EP,), jnp.int32),
              jax.ShapeDtypeStruct((EP,), jnp.int32),
              jax.ShapeDtypeStruct((NT * NB,), jnp.int32),
              jax.ShapeDtypeStruct((NT * NB,), jnp.int32),
              jax.ShapeDtypeStruct((2, NPAD), jnp.float32)),
    scratch_types=[
        pltpu.VMEM((400,), jnp.int32),
        pltpu.VMEM((128,), jnp.int32),
        pltpu.VMEM((128,), jnp.int32),
        pltpu.VMEM((128,), jnp.int32),
        pltpu.VMEM((REG,), jnp.int32),
        pltpu.VMEM((CH,), jnp.int32),
        pltpu.VMEM((CH,), jnp.int32),
        pltpu.VMEM((CH,), jnp.int32),
        pltpu.VMEM((CH,), jnp.int32),
        pltpu.VMEM((CH,), jnp.int32),
        pltpu.VMEM((CH,), jnp.float32),
        pltpu.VMEM_SHARED((NPAD,), jnp.float32),
        pltpu.SemaphoreType.DMA,
        pltpu.SemaphoreType.DMA,
        pltpu.SemaphoreType.DMA,
    ],
)
def _bin_kernel(srcP, dstP, zerosN,
                srcb, dstlocb, eidcb, cnts, loffs, degp,
                dstv, hist, curs, tagv, eidb, idx64, st_src, st_dst, st_dl,
                st_di, ones, deg_sh, sem0, sem1, sem2):
    w = _wid()
    cid = lax.axis_index("c")
    it = _iota()
    ebase = w * EPT

    @pl.when(w < 2)
    def _zero_deg():
        pltpu.sync_copy(zerosN, deg_sh)
    plsc.subcore_barrier()

    for g in range(8):
        hist[pl.ds(g * 16, 16)] = jnp.zeros((16,), jnp.int32)
    one16 = jnp.full((16,), 1, jnp.int32)

    def hist_body(i, _):
        start = pl.multiple_of(ebase + i * 400, 8)
        pltpu.sync_copy(dstP.at[pl.ds(start, 400)], dstv)
        for g in range(25):
            d16 = dstv[pl.ds(g * 16, 16)]
            plsc.addupdate_scatter(hist, [_bin_of(d16)], one16)
        return 0
    lax.fori_loop(0, 25, hist_body, 0)

    carry = jnp.int32(0)
    for g in range(4):
        hv = hist[pl.ds(g * 16, 16)]
        c8 = jnp.bitwise_and(hv + 7, -8)
        incl = plsc.cumsum(c8)
        curs[pl.ds(g * 16, 16)] = incl - c8 + carry
        carry = carry + jnp.sum(c8)
    for g in range(4, 8):
        curs[pl.ds(g * 16, 16)] = jnp.zeros((16,), jnp.int32)
    pltpu.sync_copy(hist.at[pl.ds(0, NB)], cnts.at[pl.ds(pl.multiple_of(w * NB, 8), NB)])
    pltpu.sync_copy(curs.at[pl.ds(0, NB)], loffs.at[pl.ds(pl.multiple_of(w * NB, 8), NB)])

    sent16 = jnp.full((16,), E, jnp.int32)

    def fill_body(i, _):
        eidb[pl.ds(pl.multiple_of(i * 16, 16), 16)] = sent16
        return 0
    lax.fori_loop(0, REG // 16, fill_body, 0)

    def perm_body(i, _):
        start = pl.multiple_of(ebase + i * 400, 8)
        pltpu.sync_copy(dstP.at[pl.ds(start, 400)], dstv)
        for g in range(25):
            d16 = dstv[pl.ds(g * 16, 16)]
            b16 = _bin_of(d16)
            eid16 = jnp.full((16,), ebase + i * 400 + g * 16, jnp.int32) + it

            def cond(c):
                return _extract(plsc.all_reduce_population_count(c[0])) > 0

            def body(c):
                pending = c[0]
                plsc.store_scatter(tagv, [b16], it, mask=pending)
                seen = plsc.load_gather(tagv, [b16])
                win = jnp.logical_and(pending, seen == it)
                cur = plsc.load_gather(curs, [b16])
                plsc.store_scatter(eidb, [cur], eid16, mask=win)
                plsc.addupdate_scatter(curs, [b16], one16, mask=win)
                return (jnp.logical_and(pending, jnp.logical_not(win)), 0)
            lax.while_loop(cond, body, (jnp.full((16,), True), 0))
        return 0
    lax.fori_loop(0, 25, perm_body, 0)

    for g in range(CH // 16):
        ones[pl.ds(g * 16, 16)] = jnp.full((16,), 1.0, jnp.float32)

    def out_body(j, _):
        jo = pl.multiple_of(j * CH, 8)
        for g in range(CH // 16):
            raw = eidb[pl.ds(jo + g * 16, 16)]
            idx64[pl.ds(g * 16, 16)] = jnp.minimum(raw, E - 1)
        cp0 = pltpu.async_copy(srcP.at[idx64], st_src, sem0)
        cp1 = pltpu.async_copy(dstP.at[idx64], st_dst, sem1)
        cp0.wait()
        cp1.wait()
        for g in range(CH // 16):
            sl = pl.ds(g * 16, 16)
            d16 = st_dst[sl]
            pad = eidb[pl.ds(jo + g * 16, 16)] == E
            dl = d16 - _bin_of(d16) * NPB
            st_dl[sl] = jnp.where(pad, NPB, dl)
            st_di[sl] = jnp.where(pad, NB * NPB + w, d16)
        pltpu.sync_copy(ones, deg_sh.at[st_di], add=True)
        obase = pl.multiple_of(w * REG + j * CH, 8)
        pltpu.sync_copy(st_src, srcb.at[pl.ds(obase, CH)])
        pltpu.sync_copy(st_dl, dstlocb.at[pl.ds(obase, CH)])
        pltpu.sync_copy(idx64, eidcb.at[pl.ds(obase, CH)])
        return 0
    lax.fori_loop(0, REG // CH, out_body, 0)

    plsc.subcore_barrier()

    @pl.when(w < 2)
    def _deg_out():
        pltpu.sync_copy(deg_sh, degp.at[cid])


ACC = (NPB + 1) * D


def _make_layer_kernel(last):
    nout = 2 if last else 4

    @functools.partial(
        pl.kernel, mesh=MESH, compiler_params=SC_PARAMS,
        out_type=tuple(jax.ShapeDtypeStruct((NB * NPB * D,), jnp.float32)
                       for _ in range(nout)),
        scratch_types=[pltpu.VMEM((ACC,), jnp.float32)] * nout + [
            pltpu.VMEM((CHL, D), jnp.float32),
            pltpu.VMEM((CHL, D), jnp.float32),
            pltpu.VMEM((CHL, D), jnp.float32),
            pltpu.VMEM((CHL,), jnp.int32),
            pltpu.VMEM((CHL,), jnp.int32),
            pltpu.VMEM((CHL,), jnp.int32),
            pltpu.VMEM((CHL,), jnp.int32),
            pltpu.VMEM((NT * NB,), jnp.int32),
            pltpu.VMEM((NT * NB,), jnp.int32),
            pltpu.SemaphoreType.DMA,
            pltpu.SemaphoreType.DMA,
            pltpu.SemaphoreType.DMA,
        ],
    )
    def _lk(srcb, dstlocb, eidcb, etb, psrc, pdstP, cnts, loffs, *rest):
        outs = rest[:nout]
        accs = rest[nout:2 * nout]
        (st_ps, st_pd, st_et, srcv, dstlv, gdstv, eidv, tcnt, toff,
         sem0, sem1, sem2) = rest[2 * nout:]
        if last:
            (S, MX) = outs
            (accS, accMX) = accs
        else:
            (S, Q, MX, MN) = outs
            (accS, accQ, accMX, accMN) = accs
        w = _wid()
        it = _iota()
        zero16 = jnp.zeros((16,), jnp.float32)
        neg16 = jnp.full((16,), -3e38, jnp.float32)
        pos16 = jnp.full((16,), 3e38, jnp.float32)

        pltpu.sync_copy(cnts, tcnt)
        pltpu.sync_copy(loffs, toff)

        for b_i in range(2):
            b = w * 2 + b_i

            def init_body(i, _):
                sl = pl.ds(pl.multiple_of(i * 16, 16), 16)
                accS[sl] = zero16
                accMX[sl] = neg16
                if not last:
                    accQ[sl] = zero16
                    accMN[sl] = pos16
                return 0
            lax.fori_loop(0, ACC // 16, init_body, 0)

            def region_body(r, _):
                cnt_rb = _extract(plsc.load_gather(tcnt, [jnp.full((16,), r * NB, jnp.int32) + b]))
                off_rb = _extract(plsc.load_gather(toff, [jnp.full((16,), r * NB, jnp.int32) + b]))
                base = r * REG + off_rb

                def chunk_body(ch, _):
                    start = pl.multiple_of(base + ch * CHL, 8)
                    a0 = pltpu.async_copy(srcb.at[pl.ds(start, CHL)], srcv, sem0)
                    a1 = pltpu.async_copy(dstlocb.at[pl.ds(start, CHL)], dstlv, sem1)
                    a2 = pltpu.async_copy(eidcb.at[pl.ds(start, CHL)], eidv, sem2)
                    a0.wait()
                    a2.wait()
                    b0 = pltpu.async_copy(etb.at[eidv], st_et, sem0)
                    b1 = pltpu.async_copy(psrc.at[srcv], st_ps, sem2)
                    a1.wait()
                    for g in range(CHL // 16):
                        sl = pl.ds(g * 16, 16)
                        valid = (jnp.full((16,), ch * CHL + g * 16, jnp.int32) + it) < cnt_rb
                        dl = jnp.where(valid, dstlv[sl], NPB)
                        dstlv[sl] = dl
                        gdstv[sl] = dl + b * NPB
                    b2 = pltpu.async_copy(pdstP.at[gdstv], st_pd, sem1)
                    b0.wait()
                    b1.wait()
                    b2.wait()

                    def do_edge(e):
                        ev = jnp.full((16,), e, jnp.int32)
                        rowbase = plsc.load_gather(dstlv, [ev]) * D

                        @plsc.parallel_loop(0, 8, unroll=8)
                        def gbody(g):
                            colv = jnp.full((16,), g * 16, jnp.int32) + it
                            ps = plsc.load_gather(st_ps, [ev, colv])
                            pd = plsc.load_gather(st_pd, [ev, colv])
                            et = plsc.load_gather(st_et, [ev, colv])
                            m = jnp.maximum(ps + pd + et, 0.0)
                            aidx = rowbase + colv
                            plsc.store_scatter(accS, [aidx], plsc.load_gather(accS, [aidx]) + m)
                            plsc.store_scatter(accMX, [aidx], jnp.maximum(plsc.load_gather(accMX, [aidx]), m))
                            if not last:
                                plsc.store_scatter(accQ, [aidx], plsc.load_gather(accQ, [aidx]) + m * m)
                                plsc.store_scatter(accMN, [aidx], jnp.minimum(plsc.load_gather(accMN, [aidx]), m))

                    def edge_body(e, _):
                        do_edge(e)
                        return 0
                    lax.fori_loop(0, CHL, edge_body, 0)
                    return 0
                nch = lax.div(cnt_rb + (CHL - 1), CHL)
                lax.fori_loop(0, nch, chunk_body, 0)
                return 0
            lax.fori_loop(0, NT, region_body, 0)

            obase = pl.multiple_of(b * NPB * D, 8)
            for acc, out in zip(accs, outs):
                pltpu.sync_copy(acc.at[pl.ds(0, NPB * D)], out.at[pl.ds(obase, NPB * D)])
    return _lk


_layer_full = _make_layer_kernel(False)
_layer_last = _make_layer_kernel(True)


def _mm_k(a_ref, b_ref, o_ref):
    o_ref[...] = jnp.dot(a_ref[...], b_ref[...], preferred_element_type=jnp.float32)


def _mm(a, b, block_rows):
    m, k = a.shape
    _, n = b.shape
    return pl.pallas_call(
        _mm_k,
        grid=(m // block_rows,),
        in_specs=[pl.BlockSpec((block_rows, k), lambda i: (i, 0)),
                  pl.BlockSpec((k, n), lambda i: (0, 0))],
        out_specs=pl.BlockSpec((block_rows, n), lambda i: (i, 0)),
        out_shape=jax.ShapeDtypeStruct((m, n), jnp.float32),
    )(a, b)


def _post_k(last, x_ref, s_ref, q_ref, mx_ref, mn_ref, deg_ref, w_ref, bias_ref, o_ref):
    x = x_ref[...]
    s = s_ref[...]
    deg = deg_ref[...]
    degc = jnp.maximum(deg, 1.0)
    has = (deg > 0).astype(jnp.float32)
    logd = jnp.log(deg + 1.0)
    amp = logd / DELTA
    att = DELTA / jnp.maximum(logd, 1e-6)
    mean = s / degc
    mx = mx_ref[...] * has
    if last:
        agg = jnp.concatenate([mean, mx, s], axis=1)
        feats = jnp.concatenate([x, agg, agg * amp], axis=1)
    else:
        mn = mn_ref[...] * has
        q = q_ref[...] / degc
        std = jnp.sqrt(jax.nn.relu(q - mean * mean) + 1e-5)
        agg = jnp.concatenate([mean, mx, mn, std], axis=1)
        feats = jnp.concatenate([x, agg, agg * amp, agg * att], axis=1)
    o_ref[...] = x + jnp.dot(feats, w_ref[...], preferred_element_type=jnp.float32) + bias_ref[...]


def _post(x, s, q, mx, mn, deg, wpost, bpost, last):
    br = 1000
    kdim = wpost.shape[0]
    return pl.pallas_call(
        functools.partial(_post_k, last),
        grid=(N // br,),
        in_specs=[pl.BlockSpec((br, D), lambda i: (i, 0)),
                  pl.BlockSpec((br, D), lambda i: (i, 0)),
                  pl.BlockSpec((br, D), lambda i: (i, 0)),
                  pl.BlockSpec((br, D), lambda i: (i, 0)),
                  pl.BlockSpec((br, D), lambda i: (i, 0)),
                  pl.BlockSpec((br, 1), lambda i: (i, 0)),
                  pl.BlockSpec((kdim, D), lambda i: (0, 0)),
                  pl.BlockSpec((1, D), lambda i: (0, 0))],
        out_specs=pl.BlockSpec((br, D), lambda i: (i, 0)),
        out_shape=jax.ShapeDtypeStruct((N, D), jnp.float32),
    )(x, s, q, mx, mn, deg, wpost, bpost)


def _readout_k(x_ref, w0_ref, b0_ref, w1_ref, b1_ref, w2_ref, b2_ref, o_ref):
    hg = jnp.mean(x_ref[...], axis=0, keepdims=True)
    r = jax.nn.relu(jnp.dot(hg, w0_ref[...], preferred_element_type=jnp.float32) + b0_ref[...])
    r = jax.nn.relu(jnp.dot(r, w1_ref[...], preferred_element_type=jnp.float32) + b1_ref[...])
    o_ref[...] = jnp.dot(r, w2_ref[...], preferred_element_type=jnp.float32) + b2_ref[...]


def _readout(x, w0, b0, w1, b1, w2, b2):
    return pl.pallas_call(
        _readout_k,
        out_shape=jax.ShapeDtypeStruct((1, 2), jnp.float32),
    )(x, w0, b0.reshape(1, -1), w1, b1.reshape(1, -1), w2, b2.reshape(1, -1))


def _mmb_k(a_ref, b_ref, bias_ref, o_ref):
    o_ref[...] = jnp.dot(a_ref[...], b_ref[...], preferred_element_type=jnp.float32) + bias_ref[...]


def _mm_bias(a, b, bias, block_rows):
    m, k = a.shape
    _, n = b.shape
    return pl.pallas_call(
        _mmb_k,
        grid=(m // block_rows,),
        in_specs=[pl.BlockSpec((block_rows, k), lambda i: (i, 0)),
                  pl.BlockSpec((k, n), lambda i: (0, 0)),
                  pl.BlockSpec((1, n), lambda i: (0, 0))],
        out_specs=pl.BlockSpec((block_rows, n), lambda i: (i, 0)),
        out_shape=jax.ShapeDtypeStruct((m, n), jnp.float32),
    )(a, b, bias.reshape(1, -1))


def kernel(h, e, edge_index, W_h, b_h, W_e, b_e, pre_Ws, pre_bs, post_Ws,
           post_bs, Wr0, br0, Wr1, br1, Wr2, br2):
    src = edge_index[0].astype(jnp.int32)
    dst = edge_index[1].astype(jnp.int32)
    zerosN = jnp.zeros((NPAD,), jnp.float32)

    srcb, dstlocb, eidcb, cnts, loffs, degp = _bin_kernel(src, dst, zerosN)
    deg = (degp[0] + degp[1])[:N].reshape(N, 1)

    x = _mm(h, W_h, 2000) + b_h

    for l in range(NL):
        Wpre = pre_Ws[l]
        Ws, We2, Wd = Wpre[:D], Wpre[D:2 * D], Wpre[2 * D:]
        Me = W_e @ We2
        be = b_e @ We2 + pre_bs[l]
        Psrc = _mm(x, Ws, 2000)
        Pdst = _mm(x, Wd, 2000)
        PdstP = jnp.concatenate([Pdst, jnp.zeros((NPAD - N, D), jnp.float32)], axis=0)
        Et = _mm_bias(e, Me, be, 8000)
        if l < NL - 1:
            Sf, Qf, MXf, MNf = _layer_full(srcb, dstlocb, eidcb, Et, Psrc,
                                           PdstP, cnts, loffs)
        else:
            Sf, MXf = _layer_last(srcb, dstlocb, eidcb, Et, Psrc, PdstP,
                                  cnts, loffs)
            Qf, MNf = Sf, Sf
        S = Sf.reshape(NB * NPB, D)[:N]
        Q = Qf.reshape(NB * NPB, D)[:N]
        MX = MXf.reshape(NB * NPB, D)[:N]
        MN = MNf.reshape(NB * NPB, D)[:N]
        x = _post(x, S, Q, MX, MN, deg, post_Ws[l], post_bs[l].reshape(1, D),
                  last=(l == NL - 1))

    return _readout(x, Wr0, br0, Wr1, br1, Wr2, br2)
